# baseline (speedup 1.0000x reference)
import functools

import jax
import jax.numpy as jnp
from jax import lax
from jax.experimental import pallas as pl
from jax.experimental.pallas import tpu as pltpu
from jax.experimental.pallas import tpu_sc as plsc

N_NODES = 10000
N_EDGES = 320000
D = 128
N_PAD = 10240

NC = 2
NS = 16
W_HALF = 80
CHUNK = 128
K_TILE = 160
E_PAD = NS * CHUNK * K_TILE
ROWS_2D = E_PAD // CHUNK

TC_BLK = 1024

NBUF = 2
IBUF = 4


def _make_sc_agg():
    mesh = plsc.VectorSubcoreMesh(core_axis_name="c", subcore_axis_name="s")

    @functools.partial(
        pl.kernel,
        out_type=jax.ShapeDtypeStruct((NC, N_PAD, W_HALF), jnp.float32),
        mesh=mesh,
        scratch_types=[
            pltpu.VMEM((IBUF, 1, CHUNK), jnp.int32),
            pltpu.VMEM((IBUF, 1, CHUNK), jnp.int32),
            pltpu.VMEM((NBUF, CHUNK, W_HALF), jnp.float32),
            pltpu.VMEM_SHARED((N_PAD, W_HALF), jnp.float32),
            pltpu.VMEM_SHARED((N_PAD, W_HALF), jnp.float32),
        ] + [pltpu.SemaphoreType.DMA] * (IBUF + 2 * NBUF),
        compiler_params=pltpu.CompilerParams(use_tc_tiling_on_sc=False),
        name="sc_agg_fresh1",
    )
    def sc_agg(t01, src2d, dst2d, zeros, acc, src_v, dst_v, rows_v, tab_sh,
               acc_sh, *sems):
        isem = sems[:IBUF]
        gsem = sems[IBUF:IBUF + NBUF]
        ssem = sems[IBUF + NBUF:]
        c = lax.axis_index("c")
        s = lax.axis_index("s")
        base = s * K_TILE

        def i_issue(i, q):
            pltpu.async_copy(src2d.at[pl.ds(base + i, 1)], src_v.at[q], isem[q])
            pltpu.async_copy(dst2d.at[pl.ds(base + i, 1)], dst_v.at[q], isem[q])

        def i_wait(i, q):
            pltpu.make_async_copy(src2d.at[pl.ds(base + i, 1)], src_v.at[q],
                                  isem[q]).wait()
            pltpu.make_async_copy(dst2d.at[pl.ds(base + i, 1)], dst_v.at[q],
                                  isem[q]).wait()

        def g_issue(q, b):
            pltpu.async_copy(tab_sh.at[src_v.at[q, 0]], rows_v.at[b], gsem[b])

        def g_wait(q, b):
            pltpu.make_async_copy(tab_sh.at[src_v.at[q, 0]], rows_v.at[b],
                                  gsem[b]).wait()

        def s_issue(q, b):
            pltpu.async_copy(rows_v.at[b], acc_sh.at[dst_v.at[q, 0]], ssem[b],
                             add=True)

        def s_wait(q, b):
            pltpu.make_async_copy(rows_v.at[b], acc_sh.at[dst_v.at[q, 0]],
                                  ssem[b]).wait()

        rows_per_tec = N_PAD // NS
        i_issue(0, 0)
        i_issue(1, 1)
        sl = pl.ds(s * rows_per_tec, rows_per_tec)
        pltpu.sync_copy(t01.at[c, sl], tab_sh.at[sl])
        pltpu.sync_copy(zeros.at[sl], acc_sh.at[sl])
        plsc.subcore_barrier()

        i_wait(0, 0)
        g_issue(0, 0)
        i_issue(2, 2)
        i_wait(1, 1)
        g_issue(1, 1)
        i_issue(3, 3)
        g_wait(0, 0)
        s_issue(0, 0)

        s_wait(0, 0)
        i_wait(2, 2)
        g_issue(2, 0)
        i_issue(4, 0)
        g_wait(1, 1)
        s_issue(1, 1)
        s_wait(1, 1)
        i_wait(3, 3)
        g_issue(3, 1)
        i_issue(5, 1)
        g_wait(2, 0)
        s_issue(2, 0)

        def steady(g, carry):
            i0 = 4 + g * 4
            for b in range(4):
                i = i0 + b
                q = b
                rb = b % NBUF
                s_wait((b + 2) % IBUF, rb)
                i_wait(i, q)
                g_issue(q, rb)
                @pl.when(i + 2 < K_TILE)
                def _(i=i, b=b):
                    i_issue(i + 2, (b + 2) % IBUF)
                g_wait((b + 3) % IBUF, (b + 1) % NBUF)
                s_issue((b + 3) % IBUF, (b + 1) % NBUF)
            return carry

        lax.fori_loop(0, (K_TILE - 4) // 4, steady, 0)

        g_wait(3, 1)
        s_issue(3, 1)
        s_wait(2, 0)
        s_wait(3, 1)

        plsc.subcore_barrier()

        pltpu.sync_copy(acc_sh.at[sl], acc.at[c, sl])

    return sc_agg


_SC_AGG_CACHE = []


def _sc_agg(*args):
    if not _SC_AGG_CACHE:
        _SC_AGG_CACHE.append(_make_sc_agg())
    return _SC_AGG_CACHE[0](*args)


def _unstack(two_half_block):
    feat = jnp.concatenate(
        [two_half_block[0], two_half_block[1][:, :D - W_HALF]], axis=1)
    cnt = two_half_block[1][:, D - W_HALF:D - W_HALF + 1]
    return feat, cnt


def _restack(h, valid):
    b = h.shape[0]
    ones = jnp.ones((b, 1), jnp.float32)
    pad = jnp.zeros((b, 2 * W_HALF - D - 1), jnp.float32)
    half0 = jnp.where(valid, h[:, :W_HALF], 0.0)
    half1 = jnp.where(valid,
                      jnp.concatenate([h[:, W_HALF:], ones, pad], axis=1), 0.0)
    return jnp.stack([half0, half1], axis=0)


def _tc_layer0_body(acc_ref, xa_ref, wl_ref, bl_ref, wr_ref, out_ref):
    agg, cnt = _unstack(acc_ref[...])
    mean = agg / jnp.maximum(cnt, 1.0)
    x, _ = _unstack(xa_ref[...])
    h = mean @ wl_ref[...] + bl_ref[...] + x @ wr_ref[...]
    h = jnp.maximum(h, 0.0)
    i = pl.program_id(0)
    row = i * TC_BLK + lax.broadcasted_iota(jnp.int32, (TC_BLK, 1), 0)
    out_ref[...] = _restack(h, row < N_NODES)


def _tc_layer1_body(acc_ref, ha_ref, wl_ref, bl_ref, wr_ref, wf_ref, bf_ref,
                    out_ref):
    agg, cnt = _unstack(acc_ref[...])
    mean = agg / jnp.maximum(cnt, 1.0)
    h, _ = _unstack(ha_ref[...])
    t = mean @ wl_ref[...] + bl_ref[...] + h @ wr_ref[...]
    t = jnp.maximum(t, 0.0)
    out_ref[...] = t @ wf_ref[...] + bf_ref[...]


def _tc_layer0(acc, x01, Wl0, bl0, Wr0):
    grid = (N_PAD // TC_BLK,)
    return pl.pallas_call(
        _tc_layer0_body,
        grid=grid,
        in_specs=[
            pl.BlockSpec((NC, TC_BLK, W_HALF), lambda i: (0, i, 0)),
            pl.BlockSpec((NC, TC_BLK, W_HALF), lambda i: (0, i, 0)),
            pl.BlockSpec((D, D), lambda i: (0, 0)),
            pl.BlockSpec((1, D), lambda i: (0, 0)),
            pl.BlockSpec((D, D), lambda i: (0, 0)),
        ],
        out_specs=pl.BlockSpec((NC, TC_BLK, W_HALF), lambda i: (0, i, 0)),
        out_shape=jax.ShapeDtypeStruct((NC, N_PAD, W_HALF), jnp.float32),
    )(acc, x01, Wl0, bl0, Wr0)


def _tc_layer1(acc, h01, Wl1, bl1, Wr1, Wf, bf):
    grid = (N_PAD // TC_BLK,)
    return pl.pallas_call(
        _tc_layer1_body,
        grid=grid,
        in_specs=[
            pl.BlockSpec((NC, TC_BLK, W_HALF), lambda i: (0, i, 0)),
            pl.BlockSpec((NC, TC_BLK, W_HALF), lambda i: (0, i, 0)),
            pl.BlockSpec((D, D), lambda i: (0, 0)),
            pl.BlockSpec((1, D), lambda i: (0, 0)),
            pl.BlockSpec((D, D), lambda i: (0, 0)),
            pl.BlockSpec((D, D), lambda i: (0, 0)),
            pl.BlockSpec((1, D), lambda i: (0, 0)),
        ],
        out_specs=pl.BlockSpec((TC_BLK, D), lambda i: (i, 0)),
        out_shape=jax.ShapeDtypeStruct((N_NODES, D), jnp.float32),
    )(acc, h01, Wl1, bl1, Wr1, Wf, bf)


def kernel(x, edge_index, Wl0, bl0, Wr0, Wl1, bl1, Wr1, Wf, bf):
    src = edge_index[0].astype(jnp.int32)
    dst = edge_index[1].astype(jnp.int32)
    pad_e = E_PAD - N_EDGES
    src_p = jnp.concatenate(
        [src, jnp.full((pad_e,), N_NODES, jnp.int32)]).reshape(ROWS_2D, CHUNK)
    dst_p = jnp.concatenate(
        [dst, jnp.zeros((pad_e,), jnp.int32)]).reshape(ROWS_2D, CHUNK)

    x01 = jnp.zeros((NC, N_PAD, W_HALF), jnp.float32)
    x01 = x01.at[0, :N_NODES, :].set(x[:, :W_HALF])
    x01 = x01.at[1, :N_NODES, :D - W_HALF].set(x[:, W_HALF:])
    x01 = x01.at[1, :N_NODES, D - W_HALF].set(1.0)

    zeros = jnp.zeros((N_PAD, W_HALF), jnp.float32)
    bl0r = bl0.reshape(1, D)
    bl1r = bl1.reshape(1, D)
    bfr = bf.reshape(1, D)

    acc0 = _sc_agg(x01, src_p, dst_p, zeros)
    h01 = _tc_layer0(acc0, x01, Wl0, bl0r, Wr0)
    acc1 = _sc_agg(h01, src_p, dst_p, zeros)
    return _tc_layer1(acc1, h01, Wl1, bl1r, Wr1, Wf, bfr)


# speedup vs baseline: 1.0785x; 1.0785x over previous
import functools

import jax
import jax.numpy as jnp
from jax import lax
from jax.experimental import pallas as pl
from jax.experimental.pallas import tpu as pltpu
from jax.experimental.pallas import tpu_sc as plsc

N_NODES = 10000
N_EDGES = 320000
D = 128
N_PAD = 10240

NC = 2
NS = 16
W_HALF = 80
CHUNK = 128
K_TILE = 160
E_PAD = NS * CHUNK * K_TILE
ROWS_2D = E_PAD // CHUNK

TC_BLK = 2048

NBUF = 2
IBUF = 4


def _make_sc_agg(w):
    mesh = plsc.VectorSubcoreMesh(core_axis_name="c", subcore_axis_name="s")

    @functools.partial(
        pl.kernel,
        out_type=jax.ShapeDtypeStruct((NC, N_PAD, w), jnp.float32),
        mesh=mesh,
        scratch_types=[
            pltpu.VMEM((IBUF, 1, CHUNK), jnp.int32),
            pltpu.VMEM((IBUF, 1, CHUNK), jnp.int32),
            pltpu.VMEM((NBUF, CHUNK, w), jnp.float32),
            pltpu.VMEM_SHARED((N_PAD, w), jnp.float32),
            pltpu.VMEM_SHARED((N_PAD, w), jnp.float32),
        ] + [pltpu.SemaphoreType.DMA] * (IBUF + 2 * NBUF),
        compiler_params=pltpu.CompilerParams(use_tc_tiling_on_sc=False),
        name="sc_agg_w%d" % w,
    )
    def sc_agg(t01, src2d, dst2d, zeros, acc, src_v, dst_v, rows_v, tab_sh,
               acc_sh, *sems):
        isem = sems[:IBUF]
        gsem = sems[IBUF:IBUF + NBUF]
        ssem = sems[IBUF + NBUF:]
        c = lax.axis_index("c")
        s = lax.axis_index("s")
        base = s * K_TILE

        def i_issue(i, q):
            pltpu.async_copy(src2d.at[pl.ds(base + i, 1)], src_v.at[q], isem[q])
            pltpu.async_copy(dst2d.at[pl.ds(base + i, 1)], dst_v.at[q], isem[q])

        def i_wait(i, q):
            pltpu.make_async_copy(src2d.at[pl.ds(base + i, 1)], src_v.at[q],
                                  isem[q]).wait()
            pltpu.make_async_copy(dst2d.at[pl.ds(base + i, 1)], dst_v.at[q],
                                  isem[q]).wait()

        def g_issue(q, b):
            pltpu.async_copy(tab_sh.at[src_v.at[q, 0]], rows_v.at[b], gsem[b])

        def g_wait(q, b):
            pltpu.make_async_copy(tab_sh.at[src_v.at[q, 0]], rows_v.at[b],
                                  gsem[b]).wait()

        def s_issue(q, b):
            pltpu.async_copy(rows_v.at[b], acc_sh.at[dst_v.at[q, 0]], ssem[b],
                             add=True)

        def s_wait(q, b):
            pltpu.make_async_copy(rows_v.at[b], acc_sh.at[dst_v.at[q, 0]],
                                  ssem[b]).wait()

        rows_per_tec = N_PAD // NS
        i_issue(0, 0)
        i_issue(1, 1)
        sl = pl.ds(s * rows_per_tec, rows_per_tec)
        pltpu.sync_copy(t01.at[c, sl], tab_sh.at[sl])
        pltpu.sync_copy(zeros.at[sl], acc_sh.at[sl])
        plsc.subcore_barrier()

        i_wait(0, 0)
        g_issue(0, 0)
        i_issue(2, 2)
        i_wait(1, 1)
        g_issue(1, 1)
        i_issue(3, 3)
        g_wait(0, 0)
        s_issue(0, 0)

        s_wait(0, 0)
        i_wait(2, 2)
        g_issue(2, 0)
        i_issue(4, 0)
        g_wait(1, 1)
        s_issue(1, 1)
        s_wait(1, 1)
        i_wait(3, 3)
        g_issue(3, 1)
        i_issue(5, 1)
        g_wait(2, 0)
        s_issue(2, 0)

        def steady(g, carry):
            i0 = 4 + g * 4
            for b in range(4):
                i = i0 + b
                q = b
                rb = b % NBUF
                s_wait((b + 2) % IBUF, rb)
                i_wait(i, q)
                g_issue(q, rb)
                @pl.when(i + 2 < K_TILE)
                def _(i=i, b=b):
                    i_issue(i + 2, (b + 2) % IBUF)
                g_wait((b + 3) % IBUF, (b + 1) % NBUF)
                s_issue((b + 3) % IBUF, (b + 1) % NBUF)
            return carry

        lax.fori_loop(0, (K_TILE - 4) // 4, steady, 0)

        g_wait(3, 1)
        s_issue(3, 1)
        s_wait(2, 0)
        s_wait(3, 1)

        plsc.subcore_barrier()

        pltpu.sync_copy(acc_sh.at[sl], acc.at[c, sl])

    return sc_agg


_SC_AGG_CACHE = {}


def _sc_agg(w, *args):
    if w not in _SC_AGG_CACHE:
        _SC_AGG_CACHE[w] = _make_sc_agg(w)
    return _SC_AGG_CACHE[w](*args)


W1 = 64


def _unstack80(blk):
    feat = jnp.concatenate([blk[0], blk[1][:, :D - W_HALF]], axis=1)
    cnt = blk[1][:, D - W_HALF:D - W_HALF + 1]
    return feat, cnt


def _tc_layer0_body(acc_ref, xa_ref, wl_ref, bl_ref, wr_ref, out_ref):
    agg, cnt = _unstack80(acc_ref[...])
    mean = agg / jnp.maximum(cnt, 1.0)
    x, _ = _unstack80(xa_ref[...])
    h = mean @ wl_ref[...] + bl_ref[...] + x @ wr_ref[...]
    h = jnp.maximum(h, 0.0)
    i = pl.program_id(0)
    row = i * TC_BLK + lax.broadcasted_iota(jnp.int32, (TC_BLK, 1), 0)
    valid = row < N_NODES
    half0 = jnp.where(valid, h[:, :W1], 0.0)
    half1 = jnp.where(valid, h[:, W1:], 0.0)
    out_ref[...] = jnp.stack([half0, half1], axis=0)


def _tc_layer1_body(acc1_ref, acc0_ref, ha_ref, wl_ref, bl_ref, wr_ref,
                    wf_ref, bf_ref, out_ref):
    agg = jnp.concatenate([acc1_ref[0], acc1_ref[1]], axis=1)
    cnt = acc0_ref[1][:, D - W_HALF:D - W_HALF + 1]
    mean = agg / jnp.maximum(cnt, 1.0)
    h = jnp.concatenate([ha_ref[0], ha_ref[1]], axis=1)
    t = mean @ wl_ref[...] + bl_ref[...] + h @ wr_ref[...]
    t = jnp.maximum(t, 0.0)
    out_ref[...] = t @ wf_ref[...] + bf_ref[...]


def _tc_layer0(acc, x01, Wl0, bl0, Wr0):
    grid = (N_PAD // TC_BLK,)
    return pl.pallas_call(
        _tc_layer0_body,
        grid=grid,
        in_specs=[
            pl.BlockSpec((NC, TC_BLK, W_HALF), lambda i: (0, i, 0)),
            pl.BlockSpec((NC, TC_BLK, W_HALF), lambda i: (0, i, 0)),
            pl.BlockSpec((D, D), lambda i: (0, 0)),
            pl.BlockSpec((1, D), lambda i: (0, 0)),
            pl.BlockSpec((D, D), lambda i: (0, 0)),
        ],
        out_specs=pl.BlockSpec((NC, TC_BLK, W1), lambda i: (0, i, 0)),
        out_shape=jax.ShapeDtypeStruct((NC, N_PAD, W1), jnp.float32),
    )(acc, x01, Wl0, bl0, Wr0)


def _tc_layer1(acc1, acc0, h01, Wl1, bl1, Wr1, Wf, bf):
    grid = (N_PAD // TC_BLK,)
    return pl.pallas_call(
        _tc_layer1_body,
        grid=grid,
        in_specs=[
            pl.BlockSpec((NC, TC_BLK, W1), lambda i: (0, i, 0)),
            pl.BlockSpec((NC, TC_BLK, W_HALF), lambda i: (0, i, 0)),
            pl.BlockSpec((NC, TC_BLK, W1), lambda i: (0, i, 0)),
            pl.BlockSpec((D, D), lambda i: (0, 0)),
            pl.BlockSpec((1, D), lambda i: (0, 0)),
            pl.BlockSpec((D, D), lambda i: (0, 0)),
            pl.BlockSpec((D, D), lambda i: (0, 0)),
            pl.BlockSpec((1, D), lambda i: (0, 0)),
        ],
        out_specs=pl.BlockSpec((TC_BLK, D), lambda i: (i, 0)),
        out_shape=jax.ShapeDtypeStruct((N_NODES, D), jnp.float32),
    )(acc1, acc0, h01, Wl1, bl1, Wr1, Wf, bf)


def kernel(x, edge_index, Wl0, bl0, Wr0, Wl1, bl1, Wr1, Wf, bf):
    src = edge_index[0].astype(jnp.int32)
    dst = edge_index[1].astype(jnp.int32)
    pad_e = E_PAD - N_EDGES
    src_p = jnp.concatenate(
        [src, jnp.full((pad_e,), N_NODES, jnp.int32)]).reshape(ROWS_2D, CHUNK)
    dst_p = jnp.concatenate(
        [dst, jnp.zeros((pad_e,), jnp.int32)]).reshape(ROWS_2D, CHUNK)

    x01 = jnp.zeros((NC, N_PAD, W_HALF), jnp.float32)
    x01 = x01.at[0, :N_NODES, :].set(x[:, :W_HALF])
    x01 = x01.at[1, :N_NODES, :D - W_HALF].set(x[:, W_HALF:])
    x01 = x01.at[1, :N_NODES, D - W_HALF].set(1.0)

    zeros80 = jnp.zeros((N_PAD, W_HALF), jnp.float32)
    zeros64 = jnp.zeros((N_PAD, W1), jnp.float32)
    bl0r = bl0.reshape(1, D)
    bl1r = bl1.reshape(1, D)
    bfr = bf.reshape(1, D)

    acc0 = _sc_agg(W_HALF, x01, src_p, dst_p, zeros80)
    h01 = _tc_layer0(acc0, x01, Wl0, bl0r, Wr0)
    acc1 = _sc_agg(W1, h01, src_p, dst_p, zeros64)
    return _tc_layer1(acc1, acc0, h01, Wl1, bl1r, Wr1, Wf, bfr)
